# baseline (speedup 1.0000x reference)
import jax
import jax.numpy as jnp
from jax.experimental import pallas as pl
from jax.experimental.pallas import tpu as pltpu
from jax.experimental.pallas import tpu_sc as plsc

MARGIN = 10.0
KTOP = 2
NUM_CLASSES = 64
IBLK = 1024
JBLK = 1024
RATIO = JBLK // IBLK
WCH = 128
NCH = JBLK // WCH
NEG = -1e30


SC_WORKERS = 32
SC_LANES = 16


def _sc_centers_partials(y, target):
    n, d = y.shape
    rows = n // SC_WORKERS
    mesh = plsc.VectorSubcoreMesh(core_axis_name="c", subcore_axis_name="s")

    @pl.kernel(
        out_type=jax.ShapeDtypeStruct((SC_WORKERS, NUM_CLASSES, d + SC_LANES),
                                      jnp.float32),
        mesh=mesh,
        scratch_types=[
            pltpu.VMEM((32, d), jnp.float32),
            pltpu.VMEM((rows,), jnp.int32),
            pltpu.VMEM((NUM_CLASSES, d + SC_LANES), jnp.float32),
            pltpu.SemaphoreType.DMA,
        ],
    )
    def sc_kernel(y_hbm, t_hbm, out_hbm, row_ref, tseg_ref, acc_ref, sem):
        core = jax.lax.axis_index("c")
        sub = jax.lax.axis_index("s")
        wid = sub * 2 + core
        base = wid * rows
        pltpu.async_copy(t_hbm.at[pl.ds(base, rows)], tseg_ref, sem).wait()

        @pl.loop(0, NUM_CLASSES)
        def _(cc):
            @pl.loop(0, d + SC_LANES, step=SC_LANES)
            def _(k):
                acc_ref[cc, pl.ds(k, SC_LANES)] = jnp.zeros(
                    (SC_LANES,), jnp.float32)

        @pl.loop(0, rows, step=32)
        def _(r0):
            pltpu.async_copy(y_hbm.at[pl.ds(base + r0, 32), :], row_ref,
                             sem).wait()

            @pl.loop(0, 32)
            def _(r):
                t = tseg_ref[pl.ds(r0 + r, 1)][0]

                @pl.loop(0, d, step=SC_LANES)
                def _(k):
                    acc_ref[t, pl.ds(k, SC_LANES)] = (
                        acc_ref[t, pl.ds(k, SC_LANES)]
                        + row_ref[r, pl.ds(k, SC_LANES)])
                acc_ref[t, pl.ds(d, SC_LANES)] = (
                    acc_ref[t, pl.ds(d, SC_LANES)]
                    + jnp.full((SC_LANES,), 1.0, jnp.float32))

        pltpu.async_copy(acc_ref, out_hbm.at[wid], sem).wait()

    return sc_kernel(y, target)


def _row_top2(w):
    m1 = jnp.max(w, axis=1, keepdims=True)
    eq = w == m1
    cnt = jnp.sum(jnp.where(eq, 1.0, 0.0), axis=1, keepdims=True)
    m2 = jnp.max(jnp.where(eq, NEG, w), axis=1, keepdims=True)
    return m1, jnp.where(cnt > 1.0, m1, m2)


def _loss_kernel(tr_ref, tc3_ref, ybf_ref, yt3_ref, p3_ref, out_ref,
                 rm1_ref, rm2_ref, acc1_ref, acc2_ref):
    i = pl.program_id(0)
    j = pl.program_id(1)
    n = ybf_ref.shape[0]
    c = NUM_CLASSES
    ni = n // IBLK
    nj = n // JBLK

    @pl.when(j >= i // RATIO)
    def _active():
        ybn = ybf_ref[pl.ds(i * IBLK, IBLK), :] * jnp.bfloat16(-2.0)
        ytb = yt3_ref[j]
        g = jax.lax.dot_general(ybn, ytb, (((1,), (0,)), ((), ())),
                                preferred_element_type=jnp.float32)
        ytf = ytb.astype(jnp.float32)
        ra = jnp.sum(ytf * ytf, axis=0, keepdims=True)
        t_rows = tr_ref[pl.ds(i * IBLK, IBLK), :]
        tcb = tc3_ref[j]
        colg = j * JBLK + jax.lax.broadcasted_iota(jnp.int32, (1, JBLK), 1)
        rowg = jnp.where(j == i // RATIO,
                         i * IBLK + jax.lax.broadcasted_iota(
                             jnp.int32, (IBLK, 1), 0),
                         -1)
        pairs = []
        for c0 in range(0, NCH, 2):
            w2 = []
            for c in (c0, c0 + 1):
                lo = c * WCH
                uc = g[:, lo:lo + WCH] + ra[:, lo:lo + WCH]
                mc = (t_rows == tcb[:, lo:lo + WCH]) & \
                     (colg[:, lo:lo + WCH] > rowg)
                w2.append(jnp.where(mc, uc, NEG))
            pairs.append((jnp.maximum(w2[0], w2[1]),
                          jnp.minimum(w2[0], w2[1])))
        while len(pairs) > 1:
            nxt = []
            for k in range(0, len(pairs), 2):
                (a1, a2), (b1, b2) = pairs[k], pairs[k + 1]
                nxt.append((jnp.maximum(a1, b1),
                            jnp.maximum(jnp.minimum(a1, b1),
                                        jnp.maximum(a2, b2))))
            pairs = nxt
        t1, t2 = pairs[0]

        @pl.when(j == i // RATIO)
        def _init():
            acc1_ref[...] = t1
            acc2_ref[...] = t2

        @pl.when(j > i // RATIO)
        def _merge():
            a1 = acc1_ref[...]
            a2 = acc2_ref[...]
            acc1_ref[...] = jnp.maximum(a1, t1)
            acc2_ref[...] = jnp.maximum(jnp.minimum(a1, t1),
                                        jnp.maximum(a2, t2))

    @pl.when(j == nj - 1)
    def _rowfinal():
        a = acc1_ref[...]
        m1, m2a = _row_top2(a)
        mb = jnp.max(acc2_ref[...], axis=1, keepdims=True)
        rm1_ref[pl.ds(i * IBLK, IBLK), :] = m1
        rm2_ref[pl.ds(i * IBLK, IBLK), :] = jnp.maximum(m2a, mb)

    @pl.when((i == ni - 1) & (j == nj - 1))
    def _final():
        ybf_all = ybf_ref[...].astype(jnp.float32)
        rb_all = jnp.sum(ybf_all * ybf_all, axis=1, keepdims=True)
        s1r = jnp.maximum(rm1_ref[...] + rb_all, 0.0)
        s2r = jnp.maximum(rm2_ref[...] + rb_all, 0.0)

        t_all = tr_ref[...]
        cls = jax.lax.broadcasted_iota(jnp.int32, (n, c), 1)
        oh = t_all == cls
        ohf = jnp.where(oh, 1.0, 0.0)

        w1 = jnp.where(oh, s1r, NEG)
        a1 = jnp.max(w1, axis=0, keepdims=True)
        eq1 = w1 == a1
        cnt1 = jnp.sum(jnp.where(eq1, 1.0, 0.0), axis=0, keepdims=True)
        sm1 = jnp.max(jnp.where(eq1, NEG, w1), axis=0, keepdims=True)
        sm1 = jnp.where(cnt1 > 1.0, a1, sm1)
        b1 = jnp.max(jnp.where(oh, s2r, NEG), axis=0, keepdims=True)
        top1 = jnp.maximum(a1, 0.0)
        top2 = jnp.maximum(jnp.maximum(sm1, b1), 0.0)

        cnt_row = jnp.sum(ohf, axis=0, keepdims=True)
        has_c = cnt_row > 0.0
        term = float(KTOP) / (jnp.sqrt(top1) + jnp.sqrt(top2))
        l_intra = jnp.sum(jnp.where(has_c, term, 0.0))

        d = ybf_ref.shape[1]
        part = jnp.sum(p3_ref[...], axis=0)
        cnt_col = part[:, d:d + 1]
        cen = part[:, 0:d] / cnt_col
        g2 = jax.lax.dot_general(cen, cen, (((1,), (1,)), ((), ())),
                                 preferred_element_type=jnp.float32)
        ci = jax.lax.broadcasted_iota(jnp.int32, (c, c), 0)
        cj = jax.lax.broadcasted_iota(jnp.int32, (c, c), 1)
        eyem = ci == cj
        diag_r = jnp.sum(jnp.where(eyem, g2, 0.0), axis=1, keepdims=True)
        diag_c = jnp.sum(jnp.where(eyem, g2, 0.0), axis=0, keepdims=True)
        csq = diag_r + diag_c - 2.0 * g2
        has_r = cnt_col > 0.0
        m = (cj > ci) & has_r & has_c
        dd = jnp.where(m, jnp.sqrt(jnp.maximum(csq, 0.0)), jnp.inf)
        dd = jnp.where(dd > 0.0, dd, jnp.inf)
        dmin = jnp.min(dd)
        l_inter = jnp.maximum(MARGIN - dmin, 0.0)
        out_ref[...] = jnp.reshape(l_intra + l_inter, (1, 1))


def kernel(y, target):
    n, d = y.shape
    nj = n // JBLK
    tr = target.reshape(n, 1)
    tc3 = target.reshape(nj, 1, JBLK)
    ybf = y.astype(jnp.bfloat16)
    yt3 = ybf.T.reshape(d, nj, JBLK).transpose(1, 0, 2)
    partials = _sc_centers_partials(y, target)
    out = pl.pallas_call(
        _loss_kernel,
        grid=(n // IBLK, nj),
        in_specs=[
            pl.BlockSpec((n, 1), lambda i, j: (0, 0)),
            pl.BlockSpec((nj, 1, JBLK), lambda i, j: (0, 0, 0)),
            pl.BlockSpec((n, d), lambda i, j: (0, 0)),
            pl.BlockSpec((nj, d, JBLK), lambda i, j: (0, 0, 0)),
            pl.BlockSpec((SC_WORKERS, NUM_CLASSES, d + SC_LANES),
                         lambda i, j: (0, 0, 0)),
        ],
        out_specs=pl.BlockSpec((1, 1), lambda i, j: (0, 0)),
        out_shape=jax.ShapeDtypeStruct((1, 1), jnp.float32),
        scratch_shapes=[
            pltpu.VMEM((n, 1), jnp.float32),
            pltpu.VMEM((n, 1), jnp.float32),
            pltpu.VMEM((IBLK, WCH), jnp.float32),
            pltpu.VMEM((IBLK, WCH), jnp.float32),
        ],
    )(tr, tc3, ybf, yt3, partials)
    return out.reshape(1)


# speedup vs baseline: 1.9400x; 1.9400x over previous
import jax
import jax.numpy as jnp
from jax.experimental import pallas as pl
from jax.experimental.pallas import tpu as pltpu

MARGIN = 10.0
KTOP = 2
NUM_CLASSES = 64
IBLK = 1024
JBLK = 1024
RATIO = JBLK // IBLK
WCH = 128
NCH = JBLK // WCH
NEG = -1e30


def _row_top2(w):
    m1 = jnp.max(w, axis=1, keepdims=True)
    eq = w == m1
    cnt = jnp.sum(jnp.where(eq, 1.0, 0.0), axis=1, keepdims=True)
    m2 = jnp.max(jnp.where(eq, NEG, w), axis=1, keepdims=True)
    return m1, jnp.where(cnt > 1.0, m1, m2)


def _loss_kernel(tr_ref, tc3_ref, ybf_ref, yt3_ref, out_ref,
                 rm1_ref, rm2_ref, acc1_ref, acc2_ref):
    i = pl.program_id(0)
    j = pl.program_id(1)
    n = ybf_ref.shape[0]
    c = NUM_CLASSES
    ni = n // IBLK
    nj = n // JBLK

    @pl.when(j >= i // RATIO)
    def _active():
        ybn = ybf_ref[pl.ds(i * IBLK, IBLK), :] * jnp.bfloat16(-2.0)
        ytb = yt3_ref[j]
        g = jax.lax.dot_general(ybn, ytb, (((1,), (0,)), ((), ())),
                                preferred_element_type=jnp.float32)
        ytf = ytb.astype(jnp.float32)
        ra = jnp.sum(ytf * ytf, axis=0, keepdims=True)
        t_rows = tr_ref[pl.ds(i * IBLK, IBLK), :]
        tcb = tc3_ref[j]
        colg = j * JBLK + jax.lax.broadcasted_iota(jnp.int32, (1, JBLK), 1)
        rowg = jnp.where(j == i // RATIO,
                         i * IBLK + jax.lax.broadcasted_iota(
                             jnp.int32, (IBLK, 1), 0),
                         -1)
        pairs = []
        for c0 in range(0, NCH, 2):
            w2 = []
            for c in (c0, c0 + 1):
                lo = c * WCH
                uc = g[:, lo:lo + WCH] + ra[:, lo:lo + WCH]
                mc = (t_rows == tcb[:, lo:lo + WCH]) & \
                     (colg[:, lo:lo + WCH] > rowg)
                w2.append(jnp.where(mc, uc, NEG))
            pairs.append((jnp.maximum(w2[0], w2[1]),
                          jnp.minimum(w2[0], w2[1])))
        while len(pairs) > 1:
            nxt = []
            for k in range(0, len(pairs), 2):
                (a1, a2), (b1, b2) = pairs[k], pairs[k + 1]
                nxt.append((jnp.maximum(a1, b1),
                            jnp.maximum(jnp.minimum(a1, b1),
                                        jnp.maximum(a2, b2))))
            pairs = nxt
        t1, t2 = pairs[0]

        @pl.when(j == i // RATIO)
        def _init():
            acc1_ref[...] = t1
            acc2_ref[...] = t2

        @pl.when(j > i // RATIO)
        def _merge():
            a1 = acc1_ref[...]
            a2 = acc2_ref[...]
            acc1_ref[...] = jnp.maximum(a1, t1)
            acc2_ref[...] = jnp.maximum(jnp.minimum(a1, t1),
                                        jnp.maximum(a2, t2))

    @pl.when(j == nj - 1)
    def _rowfinal():
        a = acc1_ref[...]
        m1, m2a = _row_top2(a)
        mb = jnp.max(acc2_ref[...], axis=1, keepdims=True)
        rm1_ref[pl.ds(i * IBLK, IBLK), :] = m1
        rm2_ref[pl.ds(i * IBLK, IBLK), :] = jnp.maximum(m2a, mb)

    @pl.when((i == ni - 1) & (j == nj - 1))
    def _final():
        ybf_all = ybf_ref[...].astype(jnp.float32)
        rb_all = jnp.sum(ybf_all * ybf_all, axis=1, keepdims=True)
        s1r = jnp.maximum(rm1_ref[...] + rb_all, 0.0)
        s2r = jnp.maximum(rm2_ref[...] + rb_all, 0.0)

        t_all = tr_ref[...]
        cls = jax.lax.broadcasted_iota(jnp.int32, (n, c), 1)
        oh = t_all == cls
        ohf = jnp.where(oh, 1.0, 0.0)

        w1 = jnp.where(oh, s1r, NEG)
        a1 = jnp.max(w1, axis=0, keepdims=True)
        eq1 = w1 == a1
        cnt1 = jnp.sum(jnp.where(eq1, 1.0, 0.0), axis=0, keepdims=True)
        sm1 = jnp.max(jnp.where(eq1, NEG, w1), axis=0, keepdims=True)
        sm1 = jnp.where(cnt1 > 1.0, a1, sm1)
        b1 = jnp.max(jnp.where(oh, s2r, NEG), axis=0, keepdims=True)
        top1 = jnp.maximum(a1, 0.0)
        top2 = jnp.maximum(jnp.maximum(sm1, b1), 0.0)

        cnt_row = jnp.sum(ohf, axis=0, keepdims=True)
        has_c = cnt_row > 0.0
        term = float(KTOP) / (jnp.sqrt(top1) + jnp.sqrt(top2))
        l_intra = jnp.sum(jnp.where(has_c, term, 0.0))

        y_all = ybf_all
        cen = jax.lax.dot_general(ohf, y_all, (((0,), (0,)), ((), ())),
                                  preferred_element_type=jnp.float32)
        cnt_col = jax.lax.dot_general(
            ohf, jnp.ones((n, 1), jnp.float32), (((0,), (0,)), ((), ())),
            preferred_element_type=jnp.float32)
        cen = cen / cnt_col
        g2 = jax.lax.dot_general(cen, cen, (((1,), (1,)), ((), ())),
                                 preferred_element_type=jnp.float32)
        ci = jax.lax.broadcasted_iota(jnp.int32, (c, c), 0)
        cj = jax.lax.broadcasted_iota(jnp.int32, (c, c), 1)
        eyem = ci == cj
        diag_r = jnp.sum(jnp.where(eyem, g2, 0.0), axis=1, keepdims=True)
        diag_c = jnp.sum(jnp.where(eyem, g2, 0.0), axis=0, keepdims=True)
        csq = diag_r + diag_c - 2.0 * g2
        has_r = cnt_col > 0.0
        m = (cj > ci) & has_r & has_c
        dd = jnp.where(m, jnp.sqrt(jnp.maximum(csq, 0.0)), jnp.inf)
        dd = jnp.where(dd > 0.0, dd, jnp.inf)
        dmin = jnp.min(dd)
        l_inter = jnp.maximum(MARGIN - dmin, 0.0)
        out_ref[...] = jnp.reshape(l_intra + l_inter, (1, 1))


def kernel(y, target):
    n, d = y.shape
    nj = n // JBLK
    tr = target.reshape(n, 1)
    tc3 = target.reshape(nj, 1, JBLK)
    ybf = y.astype(jnp.bfloat16)
    yt3 = ybf.T.reshape(d, nj, JBLK).transpose(1, 0, 2)
    out = pl.pallas_call(
        _loss_kernel,
        grid=(n // IBLK, nj),
        in_specs=[
            pl.BlockSpec((n, 1), lambda i, j: (0, 0)),
            pl.BlockSpec((nj, 1, JBLK), lambda i, j: (0, 0, 0)),
            pl.BlockSpec((n, d), lambda i, j: (0, 0)),
            pl.BlockSpec((nj, d, JBLK), lambda i, j: (0, 0, 0)),
        ],
        out_specs=pl.BlockSpec((1, 1), lambda i, j: (0, 0)),
        out_shape=jax.ShapeDtypeStruct((1, 1), jnp.float32),
        scratch_shapes=[
            pltpu.VMEM((n, 1), jnp.float32),
            pltpu.VMEM((n, 1), jnp.float32),
            pltpu.VMEM((IBLK, WCH), jnp.float32),
            pltpu.VMEM((IBLK, WCH), jnp.float32),
        ],
    )(tr, tc3, ybf, yt3)
    return out.reshape(1)
